# baseline (speedup 1.0000x reference)
import functools

import jax
import jax.numpy as jnp
from jax import lax
from jax.experimental import pallas as pl
from jax.experimental.pallas import tpu as pltpu
from jax.experimental.pallas import tpu_sc as plsc

N = 10000
E = 160000
D = 256
DE = 4
H = 256

NC = 2
NS = 16
NW = NC * NS

EPAD = 163840
EW_A = EPAD // NW
CA = 64
NCHUNK_A = EW_A // CA
EW_B = EPAD // NS
CB = 64
NCHUNK_B = EW_B // CB
NP = 10112
ROWS_T = NP // NS

_mesh = plsc.VectorSubcoreMesh(core_axis_name="c", subcore_axis_name="s")


def _pass_a_body(xlf, xrf, ep, srcp, dstp, att,
                 g_out, den_out,
                 att_v, den_v, g_v, tbuf,
                 ilo0, jlo0, xlr0, xrr0, er0,
                 ilo1, jlo1, xlr1, xrr1, er1,
                 semi0, semi1, semd0, semd1):
    c = lax.axis_index("c")
    s = lax.axis_index("s")
    wid = s * NC + c
    base = wid * EW_A

    pltpu.sync_copy(att, att_v)
    attv = [att_v[pl.ds(j * 16, 16)] for j in range(16)]

    def _zero(i, _):
        den_v[pl.ds(i * 16, 16)] = jnp.zeros((16,), jnp.float32)
        return 0
    lax.fori_loop(0, NP // 16, _zero, 0)

    sets = ((ilo0, jlo0, xlr0, xrr0, er0, semi0, semd0),
            (ilo1, jlo1, xlr1, xrr1, er1, semi1, semd1))

    def _issue_idx(koff, st):
        ilo, jlo, xlr, xrr, er, semi, semd = st
        pltpu.async_copy(srcp.at[pl.ds(base + koff, CA)], ilo, semi)
        pltpu.async_copy(dstp.at[pl.ds(base + koff, CA)], jlo, semi)

    def _wait_idx(st):
        ilo, jlo, xlr, xrr, er, semi, semd = st
        pltpu.make_async_copy(srcp.at[pl.ds(base, CA)], ilo, semi).wait()
        pltpu.make_async_copy(dstp.at[pl.ds(base, CA)], jlo, semi).wait()

    def _issue_data(koff, st):
        ilo, jlo, xlr, xrr, er, semi, semd = st
        pltpu.async_copy(xlf.at[ilo], xlr, semd)
        pltpu.async_copy(xrf.at[jlo], xrr, semd)
        pltpu.async_copy(ep.at[pl.ds(base + koff, CA)], er, semd)

    def _wait_data(st):
        ilo, jlo, xlr, xrr, er, semi, semd = st
        pltpu.make_async_copy(xlf.at[ilo], xlr, semd).wait()
        pltpu.make_async_copy(xrf.at[jlo], xrr, semd).wait()
        pltpu.make_async_copy(ep.at[pl.ds(base, CA)], er, semd).wait()

    lanes16 = lax.iota(jnp.int32, 16) * 16

    def _compute(koff, st):
        ilo, jlo, xlr, xrr, er, semi, semd = st

        def _group(v, _):
            goff = koff + v * 16
            for lane in range(16):
                i = v * 16 + lane
                accs = [jnp.zeros((16,), jnp.float32) for _ in range(4)]
                for j in range(16):
                    hcol = pl.ds(j * 16, 16)
                    u = (xlr[i, hcol] + xrr[i, hcol]) + er[i, hcol]
                    u = jnp.where(u > 0.0, u, u * jnp.float32(0.2))
                    accs[j % 4] = accs[j % 4] + u * attv[j]
                acc = (accs[0] + accs[1]) + (accs[2] + accs[3])
                plsc.store_scatter(tbuf, [lanes16 + lane], acc)
            alpha_acc = tbuf[pl.ds(0, 16)]
            for l in range(1, 16):
                alpha_acc = alpha_acc + tbuf[pl.ds(l * 16, 16)]
            gv = jnp.exp(alpha_acc)
            g_v[pl.ds(goff, 16)] = gv
            dstv = jlo[pl.ds(v * 16, 16)]
            plsc.addupdate_scatter(den_v, [dstv], gv)
            return 0
        lax.fori_loop(0, CA // 16, _group, 0)

    def _stage(koff, st):
        ilo, jlo = st[0], st[1]
        pltpu.sync_copy(srcp.at[pl.ds(base + koff, CA)], ilo)
        pltpu.sync_copy(dstp.at[pl.ds(base + koff, CA)], jlo)
        _issue_data(koff, st)

    _stage(0, sets[0])

    def _pair(G, _):
        k0 = G * 2 * CA
        k1 = k0 + CA
        _stage(k1, sets[1])
        _wait_data(sets[0])
        _compute(k0, sets[0])

        @pl.when(k0 + 2 * CA < EW_A)
        def _():
            _stage(k0 + 2 * CA, sets[0])
        _wait_data(sets[1])
        _compute(k1, sets[1])
        return 0
    lax.fori_loop(0, NCHUNK_A // 2, _pair, 0)

    pltpu.sync_copy(g_v, g_out.at[pl.ds(base, EW_A)])
    pltpu.sync_copy(den_v, den_out.at[pl.ds(wid * NP, NP)])


def _pass_a(xlf, xrf, ep, srcp, dstp, att):
    f = pl.kernel(
        _pass_a_body,
        out_type=[jax.ShapeDtypeStruct((EPAD,), jnp.float32),
                  jax.ShapeDtypeStruct((NW * NP,), jnp.float32)],
        mesh=_mesh,
        compiler_params=pltpu.CompilerParams(needs_layout_passes=False),
        scratch_types=[
            pltpu.VMEM((H,), jnp.float32),
            pltpu.VMEM((NP,), jnp.float32),
            pltpu.VMEM((EW_A,), jnp.float32),
            pltpu.VMEM((256,), jnp.float32),
        ] + 2 * [
            pltpu.VMEM((CA,), jnp.int32),
            pltpu.VMEM((CA,), jnp.int32),
            pltpu.VMEM((CA, H), jnp.float32),
            pltpu.VMEM((CA, H), jnp.float32),
            pltpu.VMEM((CA, H), jnp.float32),
        ] + [
            pltpu.SemaphoreType.DMA,
            pltpu.SemaphoreType.DMA,
            pltpu.SemaphoreType.DMA,
            pltpu.SemaphoreType.DMA,
        ],
    )
    return f(xlf, xrf, ep, srcp, dstp, att)


def _pass_b_body(xlh, srcp, srcph, dstp, g,
                 out,
                 sb0, db0, gb0, rows0, sb1, db1, gb1, rows1,
                 contrib, accum, semi0, semi1, semg0, semg1):
    c = lax.axis_index("c")
    s = lax.axis_index("s")
    base = s * EW_B
    r0 = s * ROWS_T

    def _zc(i, _):
        for j in range(8):
            contrib[i, pl.ds(j * 16, 16)] = jnp.zeros((16,), jnp.float32)
        return 0
    lax.fori_loop(0, CB, _zc, 0)
    nfull = ROWS_T // CB
    rem = ROWS_T - nfull * CB
    for r in range(nfull):
        pltpu.sync_copy(contrib, accum.at[pl.ds(r0 + r * CB, CB)])
    if rem:
        pltpu.sync_copy(contrib.at[pl.ds(0, rem)],
                        accum.at[pl.ds(r0 + nfull * CB, rem)])
    plsc.subcore_barrier()

    isets = ((sb0, db0, gb0, semi0), (sb1, db1, gb1, semi1))
    gsets = ((rows0, semg0), (rows1, semg1))

    def _issue_idx(koff, ist):
        sb, db, gb, sem = ist
        off = base + koff

        @pl.when(c == 0)
        def _():
            pltpu.async_copy(srcp.at[pl.ds(off, CB)], sb, sem)

        @pl.when(c == 1)
        def _():
            pltpu.async_copy(srcph.at[pl.ds(off, CB)], sb, sem)
        pltpu.async_copy(dstp.at[pl.ds(off, CB)], db, sem)
        pltpu.async_copy(g.at[pl.ds(off, CB)], gb, sem)

    def _wait_idx(ist):
        sb, db, gb, sem = ist
        pltpu.make_async_copy(srcp.at[pl.ds(base, CB)], sb, sem).wait()
        pltpu.make_async_copy(dstp.at[pl.ds(base, CB)], db, sem).wait()
        pltpu.make_async_copy(g.at[pl.ds(base, CB)], gb, sem).wait()

    def _proc(ist, gst):
        sb, db, gb, _ = ist
        rows, gsem = gst
        pltpu.make_async_copy(xlh.at[sb], rows, gsem).wait()

        def _grp(v, _):
            wv = gb[pl.ds(v * 16, 16)]
            for lane in range(16):
                i = v * 16 + lane
                ws = wv[lane]
                for j in range(8):
                    sl = pl.ds(j * 16, 16)
                    contrib[i, sl] = rows[i, sl] * ws
            return 0
        lax.fori_loop(0, CB // 16, _grp, 0)
        pltpu.sync_copy(contrib, accum.at[db], add=True)

    _issue_idx(0, isets[0])

    def _pair(G, _):
        k0 = G * 2 * CB
        k1 = k0 + CB
        _issue_idx(k1, isets[1])
        _wait_idx(isets[0])
        pltpu.async_copy(xlh.at[sb0], rows0, semg0)
        _wait_idx(isets[1])
        pltpu.async_copy(xlh.at[sb1], rows1, semg1)
        _proc(isets[0], gsets[0])

        @pl.when(k1 + CB < EW_B)
        def _():
            _issue_idx(k1 + CB, isets[0])
        _proc(isets[1], gsets[1])
        return 0
    lax.fori_loop(0, NCHUNK_B // 2, _pair, 0)

    plsc.subcore_barrier()
    pltpu.sync_copy(accum.at[pl.ds(r0, ROWS_T)],
                    out.at[pl.ds(c * NP + r0, ROWS_T)])


def _pass_b(xlh, srcp, srcph, dstp, g):
    f = pl.kernel(
        _pass_b_body,
        out_type=jax.ShapeDtypeStruct((2 * NP, 128), jnp.float32),
        mesh=_mesh,
        compiler_params=pltpu.CompilerParams(needs_layout_passes=False),
        scratch_types=2 * [
            pltpu.VMEM((CB,), jnp.int32),
            pltpu.VMEM((CB,), jnp.int32),
            pltpu.VMEM((CB,), jnp.float32),
            pltpu.VMEM((CB, 128), jnp.float32),
        ] + [
            pltpu.VMEM((CB, 128), jnp.float32),
            pltpu.VMEM_SHARED((NP, 128), jnp.float32),
            pltpu.SemaphoreType.DMA,
            pltpu.SemaphoreType.DMA,
            pltpu.SemaphoreType.DMA,
            pltpu.SemaphoreType.DMA,
        ],
    )
    return f(xlh, srcp, srcph, dstp, g)


BN = 2000


def _prep_body(x_ref, Wl_ref, bl_ref, Wr_ref, br_ref, xlh_ref, xlf_ref, xrf_ref):
    xb = x_ref[...]
    xl = xb @ Wl_ref[...] + bl_ref[...]
    xlh_ref[0] = xl
    xlf_ref[...] = xl
    xrf_ref[...] = xb @ Wr_ref[...] + br_ref[...]


_PREP_SPECS = dict(
    grid=(2, N // BN),
    out_specs=[
        pl.BlockSpec((1, BN, 128), lambda c, i: (c, i, 0)),
        pl.BlockSpec((BN, 128), lambda c, i: (i, c)),
        pl.BlockSpec((BN, 128), lambda c, i: (i, c)),
    ],
    out_shape=[jax.ShapeDtypeStruct((2, N, 128), jnp.float32),
               jax.ShapeDtypeStruct((N, H), jnp.float32),
               jax.ShapeDtypeStruct((N, H), jnp.float32)],
)


def _prep1(x, Wl, bl, Wr, br):
    out = pl.pallas_call(
        _prep_body,
        in_specs=[
            pl.BlockSpec((BN, D), lambda c, i: (i, 0)),
            pl.BlockSpec((D, 128), lambda c, i: (0, c)),
            pl.BlockSpec((128,), lambda c, i: (c,)),
            pl.BlockSpec((D, 128), lambda c, i: (0, c)),
            pl.BlockSpec((128,), lambda c, i: (c,)),
        ],
        **_PREP_SPECS,
    )(x, Wl, bl, Wr, br)
    return out[0].reshape(2 * N, 128), out[1], out[2]


def _prep2_body(g1_ref, rd_ref, b1_ref, Wl_ref, bl_ref, Wr_ref, br_ref,
                xlh_ref, xlf_ref, xrf_ref):
    h = g1_ref[...] * rd_ref[...] + b1_ref[...]
    h = jnp.where(h > 0, h, jnp.exp(h) - 1.0)
    xl = h @ Wl_ref[...] + bl_ref[...]
    xlh_ref[0] = xl
    xlf_ref[...] = xl
    xrf_ref[...] = h @ Wr_ref[...] + br_ref[...]


def _prep2(g1, rd, b1, Wl, bl, Wr, br):
    out = pl.pallas_call(
        _prep2_body,
        in_specs=[
            pl.BlockSpec((BN, H), lambda c, i: (i, 0)),
            pl.BlockSpec((BN, 1), lambda c, i: (i, 0)),
            pl.BlockSpec((H,), lambda c, i: (0,)),
            pl.BlockSpec((H, 128), lambda c, i: (0, c)),
            pl.BlockSpec((128,), lambda c, i: (c,)),
            pl.BlockSpec((H, 128), lambda c, i: (0, c)),
            pl.BlockSpec((128,), lambda c, i: (c,)),
        ],
        **_PREP_SPECS,
    )(g1, rd, b1, Wl, bl, Wr, br)
    return out[0].reshape(2 * N, 128), out[1], out[2]


BE = 2048


def _eproj_body(ea_ref, We_ref, e_ref):
    e_ref[...] = ea_ref[...] @ We_ref[...]


def _eproj(eap2, We):
    return pl.pallas_call(
        _eproj_body,
        grid=(EPAD // BE,),
        in_specs=[
            pl.BlockSpec((BE, DE), lambda i: (i, 0)),
            pl.BlockSpec((DE, H), lambda i: (0, 0)),
        ],
        out_specs=pl.BlockSpec((BE, H), lambda i: (i, 0)),
        out_shape=jax.ShapeDtypeStruct((EPAD, H), jnp.float32),
    )(eap2, We)


def _mid_body(dp_ref, rden_ref):
    s = jnp.sum(dp_ref[...], axis=0)
    rden_ref[...] = 1.0 / (s + 1e-16)


def _mid(den_part):
    return pl.pallas_call(
        _mid_body,
        out_shape=jax.ShapeDtypeStruct((NP,), jnp.float32),
    )(den_part)


def _final_body(g2_ref, rd_ref, b2_ref, x_ref, fcW_ref, fcb_ref, y_ref):
    h = g2_ref[...] * rd_ref[...] + b2_ref[...]
    h = jnp.where(h > 0, h, jnp.exp(h) - 1.0)
    y_ref[...] = (h + x_ref[...]) @ fcW_ref[...] + fcb_ref[...]


def _final(g2, rd, b2, x, fcW, fcb):
    return pl.pallas_call(
        _final_body,
        grid=(N // BN,),
        in_specs=[
            pl.BlockSpec((BN, H), lambda i: (i, 0)),
            pl.BlockSpec((BN, 1), lambda i: (i, 0)),
            pl.BlockSpec((H,), lambda i: (0,)),
            pl.BlockSpec((BN, D), lambda i: (i, 0)),
            pl.BlockSpec((H, 1), lambda i: (0, 0)),
            pl.BlockSpec((1,), lambda i: (0,)),
        ],
        out_specs=pl.BlockSpec((BN, 1), lambda i: (i, 0)),
        out_shape=jax.ShapeDtypeStruct((N, 1), jnp.float32),
    )(g2, rd, b2, x, fcW, fcb)


def _gat_layer(xlh, xlf, xrf, ep, srcp, srcph, dstp, att):
    g, den_part = _pass_a(xlf, xrf, ep, srcp, dstp, att)
    rden = _mid(den_part.reshape(NW, NP))
    o = _pass_b(xlh, srcp, srcph, dstp, g)
    num = jnp.concatenate([o[:N], o[NP:NP + N]], axis=1)
    return num, rden[:N].reshape(N, 1)


def kernel(x, edge_index, edge_attr,
           conv1_Wl, conv1_bl, conv1_Wr, conv1_br, conv1_We, conv1_att, conv1_bias,
           conv2_Wl, conv2_bl, conv2_Wr, conv2_br, conv2_We, conv2_att, conv2_bias,
           fc_W, fc_b):
    src = edge_index[0]
    dst = edge_index[1]
    npad = EPAD - E
    srcp = jnp.concatenate([src, jnp.zeros((npad,), jnp.int32)])
    srcph = srcp + N
    dstp = jnp.concatenate([dst, jnp.full((npad,), N, jnp.int32)])
    eap2 = jnp.concatenate([edge_attr, jnp.zeros((npad, DE), jnp.float32)])

    e1 = _eproj(eap2, conv1_We)
    e2 = _eproj(eap2, conv2_We)
    xlh1, xlf1, xrf1 = _prep1(x, conv1_Wl, conv1_bl, conv1_Wr, conv1_br)
    g1, rd1 = _gat_layer(xlh1, xlf1, xrf1, e1, srcp, srcph, dstp, conv1_att)
    xlh2, xlf2, xrf2 = _prep2(g1, rd1, conv1_bias, conv2_Wl, conv2_bl, conv2_Wr, conv2_br)
    g2, rd2 = _gat_layer(xlh2, xlf2, xrf2, e2, srcp, srcph, dstp, conv2_att)
    return _final(g2, rd2, conv2_bias, x, fc_W, fc_b)


# speedup vs baseline: 1.0328x; 1.0328x over previous
import functools

import jax
import jax.numpy as jnp
from jax import lax
from jax.experimental import pallas as pl
from jax.experimental.pallas import tpu as pltpu
from jax.experimental.pallas import tpu_sc as plsc

N = 10000
E = 160000
D = 256
DE = 4
H = 256

NC = 2
NS = 16
NW = NC * NS

EPAD = 163840
EW_A = EPAD // NW
CA = 64
NCHUNK_A = EW_A // CA
EW_B = EPAD // NS
CB = 64
NCHUNK_B = EW_B // CB
NP = 10112
ROWS_T = NP // NS

_mesh = plsc.VectorSubcoreMesh(core_axis_name="c", subcore_axis_name="s")


def _pass_a_body(xlf, xrf, ep, srcp, dstp, att,
                 g_out, den_out,
                 att_v, den_v, g_v, tbuf,
                 ilo0, jlo0, xlr0, xrr0, er0,
                 ilo1, jlo1, xlr1, xrr1, er1,
                 semi0, semi1, semd0, semd1):
    c = lax.axis_index("c")
    s = lax.axis_index("s")
    wid = s * NC + c
    base = wid * EW_A

    pltpu.sync_copy(att, att_v)
    attv = [att_v[pl.ds(j * 16, 16)] for j in range(16)]

    def _zero(i, _):
        den_v[pl.ds(i * 16, 16)] = jnp.zeros((16,), jnp.float32)
        return 0
    lax.fori_loop(0, NP // 16, _zero, 0)

    sets = ((ilo0, jlo0, xlr0, xrr0, er0, semi0, semd0),
            (ilo1, jlo1, xlr1, xrr1, er1, semi1, semd1))

    def _issue_idx(koff, st):
        ilo, jlo, xlr, xrr, er, semi, semd = st
        pltpu.async_copy(srcp.at[pl.ds(base + koff, CA)], ilo, semi)
        pltpu.async_copy(dstp.at[pl.ds(base + koff, CA)], jlo, semi)

    def _wait_idx(st):
        ilo, jlo, xlr, xrr, er, semi, semd = st
        pltpu.make_async_copy(srcp.at[pl.ds(base, CA)], ilo, semi).wait()
        pltpu.make_async_copy(dstp.at[pl.ds(base, CA)], jlo, semi).wait()

    def _issue_data(koff, st):
        ilo, jlo, xlr, xrr, er, semi, semd = st
        pltpu.async_copy(xlf.at[ilo], xlr, semd)
        pltpu.async_copy(xrf.at[jlo], xrr, semd)
        pltpu.async_copy(ep.at[pl.ds(base + koff, CA)], er, semd)

    def _wait_data(st):
        ilo, jlo, xlr, xrr, er, semi, semd = st
        pltpu.make_async_copy(xlf.at[ilo], xlr, semd).wait()
        pltpu.make_async_copy(xrf.at[jlo], xrr, semd).wait()
        pltpu.make_async_copy(ep.at[pl.ds(base, CA)], er, semd).wait()

    lanes16 = lax.iota(jnp.int32, 16) * 16

    def _compute(koff, st):
        ilo, jlo, xlr, xrr, er, semi, semd = st

        def _group(v, _):
            goff = koff + v * 16
            for lane in range(16):
                i = v * 16 + lane
                accs = [jnp.zeros((16,), jnp.float32) for _ in range(4)]
                for j in range(16):
                    hcol = pl.ds(j * 16, 16)
                    u = (xlr[i, hcol] + xrr[i, hcol]) + er[i, hcol]
                    u = jnp.where(u > 0.0, u, u * jnp.float32(0.2))
                    accs[j % 4] = accs[j % 4] + u * attv[j]
                acc = (accs[0] + accs[1]) + (accs[2] + accs[3])
                plsc.store_scatter(tbuf, [lanes16 + lane], acc)
            alpha_acc = tbuf[pl.ds(0, 16)]
            for l in range(1, 16):
                alpha_acc = alpha_acc + tbuf[pl.ds(l * 16, 16)]
            gv = jnp.exp(alpha_acc)
            g_v[pl.ds(goff, 16)] = gv
            dstv = jlo[pl.ds(v * 16, 16)]
            plsc.addupdate_scatter(den_v, [dstv], gv)
            return 0
        lax.fori_loop(0, CA // 16, _group, 0)

    def _stage(koff, st):
        ilo, jlo = st[0], st[1]
        pltpu.sync_copy(srcp.at[pl.ds(base + koff, CA)], ilo)
        pltpu.sync_copy(dstp.at[pl.ds(base + koff, CA)], jlo)
        _issue_data(koff, st)

    _stage(0, sets[0])

    def _pair(G, _):
        k0 = G * 2 * CA
        k1 = k0 + CA
        _stage(k1, sets[1])
        _wait_data(sets[0])
        _compute(k0, sets[0])

        @pl.when(k0 + 2 * CA < EW_A)
        def _():
            _stage(k0 + 2 * CA, sets[0])
        _wait_data(sets[1])
        _compute(k1, sets[1])
        return 0
    lax.fori_loop(0, NCHUNK_A // 2, _pair, 0)

    pltpu.sync_copy(g_v, g_out.at[pl.ds(base, EW_A)])
    pltpu.sync_copy(den_v, den_out.at[pl.ds(wid * NP, NP)])


def _pass_a(xlf, xrf, ep, srcp, dstp, att):
    f = pl.kernel(
        _pass_a_body,
        out_type=[jax.ShapeDtypeStruct((EPAD,), jnp.float32),
                  jax.ShapeDtypeStruct((NW * NP,), jnp.float32)],
        mesh=_mesh,
        compiler_params=pltpu.CompilerParams(needs_layout_passes=False),
        scratch_types=[
            pltpu.VMEM((H,), jnp.float32),
            pltpu.VMEM((NP,), jnp.float32),
            pltpu.VMEM((EW_A,), jnp.float32),
            pltpu.VMEM((256,), jnp.float32),
        ] + 2 * [
            pltpu.VMEM((CA,), jnp.int32),
            pltpu.VMEM((CA,), jnp.int32),
            pltpu.VMEM((CA, H), jnp.float32),
            pltpu.VMEM((CA, H), jnp.float32),
            pltpu.VMEM((CA, H), jnp.float32),
        ] + [
            pltpu.SemaphoreType.DMA,
            pltpu.SemaphoreType.DMA,
            pltpu.SemaphoreType.DMA,
            pltpu.SemaphoreType.DMA,
        ],
    )
    return f(xlf, xrf, ep, srcp, dstp, att)


def _pass_b_body(xlh, srcp, srcph, dstp, g,
                 out,
                 sb0, db0, gb0, rows0, ctr0, sdb0,
                 sb1, db1, gb1, rows1, ctr1, sdb1,
                 accum, semi0, semi1, semg0, semg1, semsc0, semsc1):
    c = lax.axis_index("c")
    s = lax.axis_index("s")
    base = s * EW_B
    r0 = s * ROWS_T

    def _zc(i, _):
        for j in range(8):
            z = jnp.zeros((16,), jnp.float32)
            ctr0[i, pl.ds(j * 16, 16)] = z
            ctr1[i, pl.ds(j * 16, 16)] = z
        return 0
    lax.fori_loop(0, CB, _zc, 0)
    nfull = ROWS_T // CB
    rem = ROWS_T - nfull * CB
    for r in range(nfull):
        pltpu.sync_copy(ctr0, accum.at[pl.ds(r0 + r * CB, CB)])
    if rem:
        pltpu.sync_copy(ctr0.at[pl.ds(0, rem)],
                        accum.at[pl.ds(r0 + nfull * CB, rem)])
    plsc.subcore_barrier()

    for v in range(CB // 16):
        zi = jnp.zeros((16,), jnp.int32)
        sdb0[pl.ds(v * 16, 16)] = zi
        sdb1[pl.ds(v * 16, 16)] = zi
    pltpu.async_copy(ctr0, accum.at[sdb0], semsc0, add=True)
    pltpu.async_copy(ctr1, accum.at[sdb1], semsc1, add=True)

    isets = ((sb0, db0, gb0, semi0), (sb1, db1, gb1, semi1))
    gsets = ((rows0, semg0, ctr0, sdb0, semsc0),
             (rows1, semg1, ctr1, sdb1, semsc1))

    def _issue_idx(koff, ist):
        sb, db, gb, sem = ist
        off = base + koff

        @pl.when(c == 0)
        def _():
            pltpu.async_copy(srcp.at[pl.ds(off, CB)], sb, sem)

        @pl.when(c == 1)
        def _():
            pltpu.async_copy(srcph.at[pl.ds(off, CB)], sb, sem)
        pltpu.async_copy(dstp.at[pl.ds(off, CB)], db, sem)
        pltpu.async_copy(g.at[pl.ds(off, CB)], gb, sem)

    def _wait_idx(ist):
        sb, db, gb, sem = ist
        pltpu.make_async_copy(srcp.at[pl.ds(base, CB)], sb, sem).wait()
        pltpu.make_async_copy(dstp.at[pl.ds(base, CB)], db, sem).wait()
        pltpu.make_async_copy(g.at[pl.ds(base, CB)], gb, sem).wait()

    def _proc(ist, gst):
        sb, db, gb, _ = ist
        rows, gsem, ctr, sdb, semsc = gst
        pltpu.make_async_copy(xlh.at[sb], rows, gsem).wait()
        pltpu.make_async_copy(ctr, accum.at[sdb], semsc).wait()

        def _grp(v, _):
            wv = gb[pl.ds(v * 16, 16)]
            for lane in range(16):
                i = v * 16 + lane
                ws = wv[lane]
                for j in range(8):
                    sl = pl.ds(j * 16, 16)
                    ctr[i, sl] = rows[i, sl] * ws
            return 0
        lax.fori_loop(0, CB // 16, _grp, 0)

        def _cpy(v, _):
            sl = pl.ds(v * 16, 16)
            sdb[sl] = db[sl]
            return 0
        lax.fori_loop(0, CB // 16, _cpy, 0)
        pltpu.async_copy(ctr, accum.at[sdb], semsc, add=True)

    _issue_idx(0, isets[0])

    def _pair(G, _):
        k0 = G * 2 * CB
        k1 = k0 + CB
        _issue_idx(k1, isets[1])
        _wait_idx(isets[0])
        pltpu.async_copy(xlh.at[sb0], rows0, semg0)
        _wait_idx(isets[1])
        pltpu.async_copy(xlh.at[sb1], rows1, semg1)
        _proc(isets[0], gsets[0])

        @pl.when(k1 + CB < EW_B)
        def _():
            _issue_idx(k1 + CB, isets[0])
        _proc(isets[1], gsets[1])
        return 0
    lax.fori_loop(0, NCHUNK_B // 2, _pair, 0)

    pltpu.make_async_copy(ctr0, accum.at[sdb0], semsc0).wait()
    pltpu.make_async_copy(ctr1, accum.at[sdb1], semsc1).wait()
    plsc.subcore_barrier()
    pltpu.sync_copy(accum.at[pl.ds(r0, ROWS_T)],
                    out.at[pl.ds(c * NP + r0, ROWS_T)])


def _pass_b(xlh, srcp, srcph, dstp, g):
    f = pl.kernel(
        _pass_b_body,
        out_type=jax.ShapeDtypeStruct((2 * NP, 128), jnp.float32),
        mesh=_mesh,
        compiler_params=pltpu.CompilerParams(needs_layout_passes=False),
        scratch_types=2 * [
            pltpu.VMEM((CB,), jnp.int32),
            pltpu.VMEM((CB,), jnp.int32),
            pltpu.VMEM((CB,), jnp.float32),
            pltpu.VMEM((CB, 128), jnp.float32),
            pltpu.VMEM((CB, 128), jnp.float32),
            pltpu.VMEM((CB,), jnp.int32),
        ] + [
            pltpu.VMEM_SHARED((NP, 128), jnp.float32),
            pltpu.SemaphoreType.DMA,
            pltpu.SemaphoreType.DMA,
            pltpu.SemaphoreType.DMA,
            pltpu.SemaphoreType.DMA,
            pltpu.SemaphoreType.DMA,
            pltpu.SemaphoreType.DMA,
        ],
    )
    return f(xlh, srcp, srcph, dstp, g)


BN = 2000


def _prep_body(x_ref, Wl_ref, bl_ref, Wr_ref, br_ref, xlh_ref, xlf_ref, xrf_ref):
    xb = x_ref[...]
    xl = xb @ Wl_ref[...] + bl_ref[...]
    xlh_ref[0] = xl
    xlf_ref[...] = xl
    xrf_ref[...] = xb @ Wr_ref[...] + br_ref[...]


_PREP_SPECS = dict(
    grid=(2, N // BN),
    out_specs=[
        pl.BlockSpec((1, BN, 128), lambda c, i: (c, i, 0)),
        pl.BlockSpec((BN, 128), lambda c, i: (i, c)),
        pl.BlockSpec((BN, 128), lambda c, i: (i, c)),
    ],
    out_shape=[jax.ShapeDtypeStruct((2, N, 128), jnp.float32),
               jax.ShapeDtypeStruct((N, H), jnp.float32),
               jax.ShapeDtypeStruct((N, H), jnp.float32)],
)


def _prep1(x, Wl, bl, Wr, br):
    out = pl.pallas_call(
        _prep_body,
        in_specs=[
            pl.BlockSpec((BN, D), lambda c, i: (i, 0)),
            pl.BlockSpec((D, 128), lambda c, i: (0, c)),
            pl.BlockSpec((128,), lambda c, i: (c,)),
            pl.BlockSpec((D, 128), lambda c, i: (0, c)),
            pl.BlockSpec((128,), lambda c, i: (c,)),
        ],
        **_PREP_SPECS,
    )(x, Wl, bl, Wr, br)
    return out[0].reshape(2 * N, 128), out[1], out[2]


def _prep2_body(g1_ref, rd_ref, b1_ref, Wl_ref, bl_ref, Wr_ref, br_ref,
                xlh_ref, xlf_ref, xrf_ref):
    h = g1_ref[...] * rd_ref[...] + b1_ref[...]
    h = jnp.where(h > 0, h, jnp.exp(h) - 1.0)
    xl = h @ Wl_ref[...] + bl_ref[...]
    xlh_ref[0] = xl
    xlf_ref[...] = xl
    xrf_ref[...] = h @ Wr_ref[...] + br_ref[...]


def _prep2(g1, rd, b1, Wl, bl, Wr, br):
    out = pl.pallas_call(
        _prep2_body,
        in_specs=[
            pl.BlockSpec((BN, H), lambda c, i: (i, 0)),
            pl.BlockSpec((BN, 1), lambda c, i: (i, 0)),
            pl.BlockSpec((H,), lambda c, i: (0,)),
            pl.BlockSpec((H, 128), lambda c, i: (0, c)),
            pl.BlockSpec((128,), lambda c, i: (c,)),
            pl.BlockSpec((H, 128), lambda c, i: (0, c)),
            pl.BlockSpec((128,), lambda c, i: (c,)),
        ],
        **_PREP_SPECS,
    )(g1, rd, b1, Wl, bl, Wr, br)
    return out[0].reshape(2 * N, 128), out[1], out[2]


BE = 2048


def _eproj_body(ea_ref, We_ref, e_ref):
    e_ref[...] = ea_ref[...] @ We_ref[...]


def _eproj(eap2, We):
    return pl.pallas_call(
        _eproj_body,
        grid=(EPAD // BE,),
        in_specs=[
            pl.BlockSpec((BE, DE), lambda i: (i, 0)),
            pl.BlockSpec((DE, H), lambda i: (0, 0)),
        ],
        out_specs=pl.BlockSpec((BE, H), lambda i: (i, 0)),
        out_shape=jax.ShapeDtypeStruct((EPAD, H), jnp.float32),
    )(eap2, We)


def _mid_body(dp_ref, rden_ref):
    s = jnp.sum(dp_ref[...], axis=0)
    rden_ref[...] = 1.0 / (s + 1e-16)


def _mid(den_part):
    return pl.pallas_call(
        _mid_body,
        out_shape=jax.ShapeDtypeStruct((NP,), jnp.float32),
    )(den_part)


def _final_body(g2_ref, rd_ref, b2_ref, x_ref, fcW_ref, fcb_ref, y_ref):
    h = g2_ref[...] * rd_ref[...] + b2_ref[...]
    h = jnp.where(h > 0, h, jnp.exp(h) - 1.0)
    y_ref[...] = (h + x_ref[...]) @ fcW_ref[...] + fcb_ref[...]


def _final(g2, rd, b2, x, fcW, fcb):
    return pl.pallas_call(
        _final_body,
        grid=(N // BN,),
        in_specs=[
            pl.BlockSpec((BN, H), lambda i: (i, 0)),
            pl.BlockSpec((BN, 1), lambda i: (i, 0)),
            pl.BlockSpec((H,), lambda i: (0,)),
            pl.BlockSpec((BN, D), lambda i: (i, 0)),
            pl.BlockSpec((H, 1), lambda i: (0, 0)),
            pl.BlockSpec((1,), lambda i: (0,)),
        ],
        out_specs=pl.BlockSpec((BN, 1), lambda i: (i, 0)),
        out_shape=jax.ShapeDtypeStruct((N, 1), jnp.float32),
    )(g2, rd, b2, x, fcW, fcb)


def _gat_layer(xlh, xlf, xrf, ep, srcp, srcph, dstp, att):
    g, den_part = _pass_a(xlf, xrf, ep, srcp, dstp, att)
    rden = _mid(den_part.reshape(NW, NP))
    o = _pass_b(xlh, srcp, srcph, dstp, g)
    num = jnp.concatenate([o[:N], o[NP:NP + N]], axis=1)
    return num, rden[:N].reshape(N, 1)


def kernel(x, edge_index, edge_attr,
           conv1_Wl, conv1_bl, conv1_Wr, conv1_br, conv1_We, conv1_att, conv1_bias,
           conv2_Wl, conv2_bl, conv2_Wr, conv2_br, conv2_We, conv2_att, conv2_bias,
           fc_W, fc_b):
    src = edge_index[0]
    dst = edge_index[1]
    npad = EPAD - E
    srcp = jnp.concatenate([src, jnp.zeros((npad,), jnp.int32)])
    srcph = srcp + N
    dstp = jnp.concatenate([dst, jnp.full((npad,), N, jnp.int32)])
    eap2 = jnp.concatenate([edge_attr, jnp.zeros((npad, DE), jnp.float32)])

    e1 = _eproj(eap2, conv1_We)
    e2 = _eproj(eap2, conv2_We)
    xlh1, xlf1, xrf1 = _prep1(x, conv1_Wl, conv1_bl, conv1_Wr, conv1_br)
    g1, rd1 = _gat_layer(xlh1, xlf1, xrf1, e1, srcp, srcph, dstp, conv1_att)
    xlh2, xlf2, xrf2 = _prep2(g1, rd1, conv1_bias, conv2_Wl, conv2_bl, conv2_Wr, conv2_br)
    g2, rd2 = _gat_layer(xlh2, xlf2, xrf2, e2, srcp, srcph, dstp, conv2_att)
    return _final(g2, rd2, conv2_bias, x, fc_W, fc_b)


# speedup vs baseline: 1.0331x; 1.0003x over previous
import jax
import jax.numpy as jnp
from jax import lax
from jax.experimental import pallas as pl
from jax.experimental.pallas import tpu as pltpu
from jax.experimental.pallas import tpu_sc as plsc

N = 10000
E = 160000
D = 256
DE = 4
H = 256

NC = 2
NS = 16
NW = NC * NS

EPAD = 163840
EW_A = EPAD // NW
CA = 64
NCHUNK_A = EW_A // CA
EW_B = EPAD // NS
CB = 64
NCHUNK_B = EW_B // CB
NP = 10112
ROWS_T = NP // NS

_mesh = plsc.VectorSubcoreMesh(core_axis_name="c", subcore_axis_name="s")


def _pass_a_body(xlf, xrf, ep, srcp, dstp, att,
                 g_out, den_out,
                 att_v, den_v, g_v, tbuf,
                 ilo0, jlo0, xlr0, xrr0, er0,
                 ilo1, jlo1, xlr1, xrr1, er1,
                 semi0, semi1, semd0, semd1):
    c = lax.axis_index("c")
    s = lax.axis_index("s")
    wid = s * NC + c
    base = wid * EW_A

    pltpu.sync_copy(att, att_v)
    attv = [att_v[pl.ds(j * 16, 16)] for j in range(16)]

    def _zero(i, _):
        den_v[pl.ds(i * 16, 16)] = jnp.zeros((16,), jnp.float32)
        return 0
    lax.fori_loop(0, NP // 16, _zero, 0)

    sets = ((ilo0, jlo0, xlr0, xrr0, er0, semi0, semd0),
            (ilo1, jlo1, xlr1, xrr1, er1, semi1, semd1))

    def _issue_idx(koff, st):
        ilo, jlo, xlr, xrr, er, semi, semd = st
        pltpu.async_copy(srcp.at[pl.ds(base + koff, CA)], ilo, semi)
        pltpu.async_copy(dstp.at[pl.ds(base + koff, CA)], jlo, semi)

    def _wait_idx(st):
        ilo, jlo, xlr, xrr, er, semi, semd = st
        pltpu.make_async_copy(srcp.at[pl.ds(base, CA)], ilo, semi).wait()
        pltpu.make_async_copy(dstp.at[pl.ds(base, CA)], jlo, semi).wait()

    def _issue_data(koff, st):
        ilo, jlo, xlr, xrr, er, semi, semd = st
        pltpu.async_copy(xlf.at[ilo], xlr, semd)
        pltpu.async_copy(xrf.at[jlo], xrr, semd)
        pltpu.async_copy(ep.at[pl.ds(base + koff, CA)], er, semd)

    def _wait_data(st):
        ilo, jlo, xlr, xrr, er, semi, semd = st
        pltpu.make_async_copy(xlf.at[ilo], xlr, semd).wait()
        pltpu.make_async_copy(xrf.at[jlo], xrr, semd).wait()
        pltpu.make_async_copy(ep.at[pl.ds(base, CA)], er, semd).wait()

    lanes16 = lax.iota(jnp.int32, 16) * 16

    def _compute(koff, st):
        ilo, jlo, xlr, xrr, er, semi, semd = st

        def _group(v, _):
            goff = koff + v * 16
            for lane in range(16):
                i = v * 16 + lane
                accs = [jnp.zeros((16,), jnp.float32) for _ in range(4)]
                for j in range(16):
                    hcol = pl.ds(j * 16, 16)
                    u = (xlr[i, hcol] + xrr[i, hcol]) + er[i, hcol]
                    u = jnp.where(u > 0.0, u, u * jnp.float32(0.2))
                    accs[j % 4] = accs[j % 4] + u * attv[j]
                acc = (accs[0] + accs[1]) + (accs[2] + accs[3])
                plsc.store_scatter(tbuf, [lanes16 + lane], acc)
            alpha_acc = tbuf[pl.ds(0, 16)]
            for l in range(1, 16):
                alpha_acc = alpha_acc + tbuf[pl.ds(l * 16, 16)]
            gv = jnp.exp(alpha_acc)
            g_v[pl.ds(goff, 16)] = gv
            dstv = jlo[pl.ds(v * 16, 16)]
            plsc.addupdate_scatter(den_v, [dstv], gv)
            return 0
        lax.fori_loop(0, CA // 16, _group, 0)

    def _stage(koff, st):
        ilo, jlo = st[0], st[1]
        pltpu.sync_copy(srcp.at[pl.ds(base + koff, CA)], ilo)
        pltpu.sync_copy(dstp.at[pl.ds(base + koff, CA)], jlo)
        _issue_data(koff, st)

    _stage(0, sets[0])

    def _pair(G, _):
        k0 = G * 2 * CA
        k1 = k0 + CA
        _stage(k1, sets[1])
        _wait_data(sets[0])
        _compute(k0, sets[0])

        @pl.when(k0 + 2 * CA < EW_A)
        def _():
            _stage(k0 + 2 * CA, sets[0])
        _wait_data(sets[1])
        _compute(k1, sets[1])
        return 0
    lax.fori_loop(0, NCHUNK_A // 2, _pair, 0)

    pltpu.sync_copy(g_v, g_out.at[pl.ds(base, EW_A)])
    pltpu.sync_copy(den_v, den_out.at[pl.ds(wid * NP, NP)])


def _pass_a(xlf, xrf, ep, srcp, dstp, att):
    f = pl.kernel(
        _pass_a_body,
        out_type=[jax.ShapeDtypeStruct((EPAD,), jnp.float32),
                  jax.ShapeDtypeStruct((NW * NP,), jnp.float32)],
        mesh=_mesh,
        compiler_params=pltpu.CompilerParams(needs_layout_passes=False),
        scratch_types=[
            pltpu.VMEM((H,), jnp.float32),
            pltpu.VMEM((NP,), jnp.float32),
            pltpu.VMEM((EW_A,), jnp.float32),
            pltpu.VMEM((256,), jnp.float32),
        ] + 2 * [
            pltpu.VMEM((CA,), jnp.int32),
            pltpu.VMEM((CA,), jnp.int32),
            pltpu.VMEM((CA, H), jnp.float32),
            pltpu.VMEM((CA, H), jnp.float32),
            pltpu.VMEM((CA, H), jnp.float32),
        ] + [
            pltpu.SemaphoreType.DMA,
            pltpu.SemaphoreType.DMA,
            pltpu.SemaphoreType.DMA,
            pltpu.SemaphoreType.DMA,
        ],
    )
    return f(xlf, xrf, ep, srcp, dstp, att)


def _pass_b_body(xlh, srcp, srcph, dstp, g,
                 out,
                 sb0, db0, gb0, rows0, ctr0, sdb0,
                 sb1, db1, gb1, rows1, ctr1, sdb1,
                 accum, semi0, semi1, semg0, semg1, semsc0, semsc1):
    c = lax.axis_index("c")
    s = lax.axis_index("s")
    base = s * EW_B
    r0 = s * ROWS_T

    def _zc(i, _):
        for j in range(8):
            z = jnp.zeros((16,), jnp.float32)
            ctr0[i, pl.ds(j * 16, 16)] = z
            ctr1[i, pl.ds(j * 16, 16)] = z
        return 0
    lax.fori_loop(0, CB, _zc, 0)
    nfull = ROWS_T // CB
    rem = ROWS_T - nfull * CB
    for r in range(nfull):
        pltpu.sync_copy(ctr0, accum.at[pl.ds(r0 + r * CB, CB)])
    if rem:
        pltpu.sync_copy(ctr0.at[pl.ds(0, rem)],
                        accum.at[pl.ds(r0 + nfull * CB, rem)])
    plsc.subcore_barrier()

    for v in range(CB // 16):
        zi = jnp.zeros((16,), jnp.int32)
        sdb0[pl.ds(v * 16, 16)] = zi
        sdb1[pl.ds(v * 16, 16)] = zi
    pltpu.async_copy(ctr0, accum.at[sdb0], semsc0, add=True)
    pltpu.async_copy(ctr1, accum.at[sdb1], semsc1, add=True)

    isets = ((sb0, db0, gb0, semi0), (sb1, db1, gb1, semi1))
    gsets = ((rows0, semg0, ctr0, sdb0, semsc0),
             (rows1, semg1, ctr1, sdb1, semsc1))

    def _issue_idx(koff, ist):
        sb, db, gb, sem = ist
        off = base + koff

        @pl.when(c == 0)
        def _():
            pltpu.async_copy(srcp.at[pl.ds(off, CB)], sb, sem)

        @pl.when(c == 1)
        def _():
            pltpu.async_copy(srcph.at[pl.ds(off, CB)], sb, sem)
        pltpu.async_copy(dstp.at[pl.ds(off, CB)], db, sem)
        pltpu.async_copy(g.at[pl.ds(off, CB)], gb, sem)

    def _wait_idx(ist):
        sb, db, gb, sem = ist
        pltpu.make_async_copy(srcp.at[pl.ds(base, CB)], sb, sem).wait()
        pltpu.make_async_copy(dstp.at[pl.ds(base, CB)], db, sem).wait()
        pltpu.make_async_copy(g.at[pl.ds(base, CB)], gb, sem).wait()

    def _proc(ist, gst):
        sb, db, gb, _ = ist
        rows, gsem, ctr, sdb, semsc = gst
        pltpu.make_async_copy(xlh.at[sb], rows, gsem).wait()
        pltpu.make_async_copy(ctr, accum.at[sdb], semsc).wait()

        def _grp(v, _):
            wv = gb[pl.ds(v * 16, 16)]
            for lane in range(16):
                i = v * 16 + lane
                ws = wv[lane]
                for j in range(8):
                    sl = pl.ds(j * 16, 16)
                    ctr[i, sl] = rows[i, sl] * ws
            return 0
        lax.fori_loop(0, CB // 16, _grp, 0)

        def _cpy(v, _):
            sl = pl.ds(v * 16, 16)
            sdb[sl] = db[sl]
            return 0
        lax.fori_loop(0, CB // 16, _cpy, 0)
        pltpu.async_copy(ctr, accum.at[sdb], semsc, add=True)

    _issue_idx(0, isets[0])

    def _pair(G, _):
        k0 = G * 2 * CB
        k1 = k0 + CB
        _issue_idx(k1, isets[1])
        _wait_idx(isets[0])
        pltpu.async_copy(xlh.at[sb0], rows0, semg0)
        _wait_idx(isets[1])
        pltpu.async_copy(xlh.at[sb1], rows1, semg1)
        _proc(isets[0], gsets[0])

        @pl.when(k1 + CB < EW_B)
        def _():
            _issue_idx(k1 + CB, isets[0])
        _proc(isets[1], gsets[1])
        return 0
    lax.fori_loop(0, NCHUNK_B // 2, _pair, 0)

    pltpu.make_async_copy(ctr0, accum.at[sdb0], semsc0).wait()
    pltpu.make_async_copy(ctr1, accum.at[sdb1], semsc1).wait()
    plsc.subcore_barrier()
    pltpu.sync_copy(accum.at[pl.ds(r0, ROWS_T)],
                    out.at[pl.ds(c * NP + r0, ROWS_T)])


def _pass_b(xlh, srcp, srcph, dstp, g):
    f = pl.kernel(
        _pass_b_body,
        out_type=jax.ShapeDtypeStruct((2 * NP, 128), jnp.float32),
        mesh=_mesh,
        compiler_params=pltpu.CompilerParams(needs_layout_passes=False),
        scratch_types=2 * [
            pltpu.VMEM((CB,), jnp.int32),
            pltpu.VMEM((CB,), jnp.int32),
            pltpu.VMEM((CB,), jnp.float32),
            pltpu.VMEM((CB, 128), jnp.float32),
            pltpu.VMEM((CB, 128), jnp.float32),
            pltpu.VMEM((CB,), jnp.int32),
        ] + [
            pltpu.VMEM_SHARED((NP, 128), jnp.float32),
            pltpu.SemaphoreType.DMA,
            pltpu.SemaphoreType.DMA,
            pltpu.SemaphoreType.DMA,
            pltpu.SemaphoreType.DMA,
            pltpu.SemaphoreType.DMA,
            pltpu.SemaphoreType.DMA,
        ],
    )
    return f(xlh, srcp, srcph, dstp, g)


BN = 2000


def _prep_body(x_ref, Wl_ref, bl_ref, Wr_ref, br_ref, xlh_ref, xlf_ref, xrf_ref):
    xb = x_ref[...]
    xl = xb @ Wl_ref[...] + bl_ref[...]
    xlh_ref[0] = xl
    xlf_ref[...] = xl
    xrf_ref[...] = xb @ Wr_ref[...] + br_ref[...]


_PREP_SPECS = dict(
    grid=(2, N // BN),
    out_specs=[
        pl.BlockSpec((1, BN, 128), lambda c, i: (c, i, 0)),
        pl.BlockSpec((BN, 128), lambda c, i: (i, c)),
        pl.BlockSpec((BN, 128), lambda c, i: (i, c)),
    ],
    out_shape=[jax.ShapeDtypeStruct((2, N, 128), jnp.float32),
               jax.ShapeDtypeStruct((N, H), jnp.float32),
               jax.ShapeDtypeStruct((N, H), jnp.float32)],
)


def _prep1(x, Wl, bl, Wr, br):
    out = pl.pallas_call(
        _prep_body,
        in_specs=[
            pl.BlockSpec((BN, D), lambda c, i: (i, 0)),
            pl.BlockSpec((D, 128), lambda c, i: (0, c)),
            pl.BlockSpec((128,), lambda c, i: (c,)),
            pl.BlockSpec((D, 128), lambda c, i: (0, c)),
            pl.BlockSpec((128,), lambda c, i: (c,)),
        ],
        **_PREP_SPECS,
    )(x, Wl, bl, Wr, br)
    return out[0].reshape(2 * N, 128), out[1], out[2]


def _prep2_body(g1_ref, rd_ref, b1_ref, Wl_ref, bl_ref, Wr_ref, br_ref,
                xlh_ref, xlf_ref, xrf_ref):
    h = g1_ref[...] * rd_ref[...] + b1_ref[...]
    h = jnp.where(h > 0, h, jnp.exp(h) - 1.0)
    xl = h @ Wl_ref[...] + bl_ref[...]
    xlh_ref[0] = xl
    xlf_ref[...] = xl
    xrf_ref[...] = h @ Wr_ref[...] + br_ref[...]


def _prep2(g1, rd, b1, Wl, bl, Wr, br):
    out = pl.pallas_call(
        _prep2_body,
        in_specs=[
            pl.BlockSpec((BN, H), lambda c, i: (i, 0)),
            pl.BlockSpec((BN, 1), lambda c, i: (i, 0)),
            pl.BlockSpec((H,), lambda c, i: (0,)),
            pl.BlockSpec((H, 128), lambda c, i: (0, c)),
            pl.BlockSpec((128,), lambda c, i: (c,)),
            pl.BlockSpec((H, 128), lambda c, i: (0, c)),
            pl.BlockSpec((128,), lambda c, i: (c,)),
        ],
        **_PREP_SPECS,
    )(g1, rd, b1, Wl, bl, Wr, br)
    return out[0].reshape(2 * N, 128), out[1], out[2]


BE = 2048


def _eproj_body(ea_ref, We_ref, e_ref):
    e_ref[...] = ea_ref[...] @ We_ref[...]


def _eproj(eap2, We):
    return pl.pallas_call(
        _eproj_body,
        grid=(EPAD // BE,),
        in_specs=[
            pl.BlockSpec((BE, DE), lambda i: (i, 0)),
            pl.BlockSpec((DE, H), lambda i: (0, 0)),
        ],
        out_specs=pl.BlockSpec((BE, H), lambda i: (i, 0)),
        out_shape=jax.ShapeDtypeStruct((EPAD, H), jnp.float32),
    )(eap2, We)


def _mid_body(dp_ref, rden_ref):
    s = jnp.sum(dp_ref[...], axis=0)
    rden_ref[...] = 1.0 / (s + 1e-16)


def _mid(den_part):
    return pl.pallas_call(
        _mid_body,
        out_shape=jax.ShapeDtypeStruct((NP,), jnp.float32),
    )(den_part)


def _final_body(g2_ref, rd_ref, b2_ref, x_ref, fcW_ref, fcb_ref, y_ref):
    h = g2_ref[...] * rd_ref[...] + b2_ref[...]
    h = jnp.where(h > 0, h, jnp.exp(h) - 1.0)
    y_ref[...] = (h + x_ref[...]) @ fcW_ref[...] + fcb_ref[...]


def _final(g2, rd, b2, x, fcW, fcb):
    return pl.pallas_call(
        _final_body,
        grid=(N // BN,),
        in_specs=[
            pl.BlockSpec((BN, H), lambda i: (i, 0)),
            pl.BlockSpec((BN, 1), lambda i: (i, 0)),
            pl.BlockSpec((H,), lambda i: (0,)),
            pl.BlockSpec((BN, D), lambda i: (i, 0)),
            pl.BlockSpec((H, 1), lambda i: (0, 0)),
            pl.BlockSpec((1,), lambda i: (0,)),
        ],
        out_specs=pl.BlockSpec((BN, 1), lambda i: (i, 0)),
        out_shape=jax.ShapeDtypeStruct((N, 1), jnp.float32),
    )(g2, rd, b2, x, fcW, fcb)


def _gat_layer(xlh, xlf, xrf, ep, srcp, srcph, dstp, att):
    g, den_part = _pass_a(xlf, xrf, ep, srcp, dstp, att)
    rden = _mid(den_part.reshape(NW, NP))
    o = _pass_b(xlh, srcp, srcph, dstp, g)
    num = jnp.concatenate([o[:N], o[NP:NP + N]], axis=1)
    return num, rden[:N].reshape(N, 1)


def kernel(x, edge_index, edge_attr,
           conv1_Wl, conv1_bl, conv1_Wr, conv1_br, conv1_We, conv1_att, conv1_bias,
           conv2_Wl, conv2_bl, conv2_Wr, conv2_br, conv2_We, conv2_att, conv2_bias,
           fc_W, fc_b):
    src = edge_index[0]
    dst = edge_index[1]
    npad = EPAD - E
    srcp = jnp.concatenate([src, jnp.zeros((npad,), jnp.int32)])
    srcph = srcp + N
    dstp = jnp.concatenate([dst, jnp.full((npad,), N, jnp.int32)])
    eap2 = jnp.concatenate([edge_attr, jnp.zeros((npad, DE), jnp.float32)])

    e1 = _eproj(eap2, conv1_We)
    e2 = _eproj(eap2, conv2_We)
    xlh1, xlf1, xrf1 = _prep1(x, conv1_Wl, conv1_bl, conv1_Wr, conv1_br)
    g1, rd1 = _gat_layer(xlh1, xlf1, xrf1, e1, srcp, srcph, dstp, conv1_att)
    xlh2, xlf2, xrf2 = _prep2(g1, rd1, conv1_bias, conv2_Wl, conv2_bl, conv2_Wr, conv2_br)
    g2, rd2 = _gat_layer(xlh2, xlf2, xrf2, e2, srcp, srcph, dstp, conv2_att)
    return _final(g2, rd2, conv2_bias, x, fc_W, fc_b)
